# baseline (speedup 1.0000x reference)
import jax
import jax.numpy as jnp
import numpy as np
from jax.experimental import pallas as pl

WS = 8
B, H, W = 16, 512, 512
GH, GW = H // WS, W // WS
CELL = WS * WS
PER_BATCH_CAT = GH * GW * CELL
PER_BATCH_BERN = GH * GW

K1 = (np.uint32(928981903), np.uint32(3453687069))
K2 = (np.uint32(4146024105), np.uint32(2718843009))
TINY = np.float32(np.finfo(np.float32).tiny)


def _rotl(x, d):
    return (x << np.uint32(d)) | (x >> np.uint32(32 - d))


def _threefry_bits(key, x1):
    k0, k1 = key
    ks = (k0, k1, np.uint32(np.uint32(k0) ^ np.uint32(k1) ^ np.uint32(0x1BD11BDA)))
    rot = ((13, 15, 26, 6), (17, 29, 16, 24))
    x0 = jnp.full_like(x1, ks[0])
    x1 = x1 + ks[1]
    for i in range(5):
        for r in rot[i % 2]:
            x0 = x0 + x1
            x1 = _rotl(x1, r) ^ x0
        x0 = x0 + ks[(i + 1) % 3]
        x1 = x1 + ks[(i + 2) % 3] + np.uint32(i + 1)
    return x0 ^ x1


def _u01(bits):
    f = jax.lax.bitcast_convert_type(
        (bits >> np.uint32(9)) | np.uint32(0x3F800000), jnp.float32)
    return f - jnp.float32(1.0)


def _log_sigmoid(x):
    return jnp.minimum(x, 0.0) - jnp.log1p(jnp.exp(-jnp.abs(x)))


def _body(x_ref, lp_ref, acc_ref, xf_ref, yf_ref):
    b = pl.program_id(0).astype(jnp.uint32)
    img = x_ref[0]
    cells = jnp.transpose(
        img.reshape(GH, WS, GW, WS), (0, 2, 1, 3)).reshape(GH, GW, CELL)

    shp = (GH, GW, CELL)
    n = (jax.lax.broadcasted_iota(jnp.uint32, shp, 0) * np.uint32(GW * CELL)
         + jax.lax.broadcasted_iota(jnp.uint32, shp, 1) * np.uint32(CELL)
         + jax.lax.broadcasted_iota(jnp.uint32, shp, 2)
         + b * np.uint32(PER_BATCH_CAT))
    u = _u01(_threefry_bits(K1, n)) + TINY
    score = cells - jnp.log(-jnp.log(u))

    lanes = jax.lax.broadcasted_iota(jnp.int32, shp, 2)
    mx = jnp.max(score, axis=-1, keepdims=True)
    choice = jnp.min(jnp.where(score == mx, lanes, CELL), axis=-1)
    chm = lanes == choice[..., None]

    selected = jnp.sum(jnp.where(chm, cells, 0.0), axis=-1)
    xmax = jnp.max(cells, axis=-1, keepdims=True)
    shifted = cells - xmax
    logp_cat = (selected - xmax[..., 0]) - jnp.log(jnp.sum(jnp.exp(shifted), axis=-1))

    shp2 = (GH, GW)
    n2 = (jax.lax.broadcasted_iota(jnp.uint32, shp2, 0) * np.uint32(GW)
          + jax.lax.broadcasted_iota(jnp.uint32, shp2, 1)
          + b * np.uint32(PER_BATCH_BERN))
    u2 = _u01(_threefry_bits(K2, n2))
    p = jax.nn.sigmoid(selected)
    acc = (u2 < p).astype(jnp.float32)

    logp_bern = acc * _log_sigmoid(selected) + (1.0 - acc) * _log_sigmoid(-selected)
    lp_ref[0] = logp_cat + logp_bern
    acc_ref[0] = acc

    gi = jax.lax.broadcasted_iota(jnp.int32, shp2, 0)
    gj = jax.lax.broadcasted_iota(jnp.int32, shp2, 1)
    xf_ref[0] = (gj * WS + (choice & 7)).astype(jnp.float32)
    yf_ref[0] = (gi * WS + (choice >> 3)).astype(jnp.float32)


def _run(x, interpret=False):
    xr = x.reshape(B, H, W)
    out = jax.ShapeDtypeStruct((B, GH, GW), jnp.float32)
    ospec = pl.BlockSpec((1, GH, GW), lambda b: (b, 0, 0))
    lp, acc, xf, yf = pl.pallas_call(
        _body,
        grid=(B,),
        in_specs=[pl.BlockSpec((1, H, W), lambda b: (b, 0, 0))],
        out_specs=[ospec, ospec, ospec, ospec],
        out_shape=[out, out, out, out],
        interpret=interpret,
    )(xr)
    xy = jnp.stack([xf, yf], axis=-1)
    return xy, lp, acc > 0


def kernel(x):
    return _run(x)
